# baseline (speedup 1.0000x reference)
import functools

import jax
import jax.numpy as jnp
from jax import lax
from jax.experimental import pallas as pl
from jax.experimental.pallas import tpu as pltpu
from jax.experimental.pallas import tpu_sc as plsc

_N = 10000
_D = 128
_E = 320000
_NC = 2
_NS = 16
_NW = _NC * _NS
_EPW = _E // _NW
_CHUNK = 128
_NCHUNK = 80
_EPWP = _NCHUNK * _CHUNK
_NP = 10240
_RPT = _NP // _NS

_BLK = 2000

def _unpack_chunk(slab_v, j, src_b, dst_b):
    row = slab_v.at[j]
    for k in range(_CHUNK // 16):
        e = row[pl.ds(16 * k, 16)]
        src_b[pl.ds(16 * k, 16)] = e & 0xFFFF
        dst_b[pl.ds(16 * k, 16)] = lax.shift_right_logical(e, 16)


def _agg_body(h_hbm, eidx_hbm, zero_hbm, out_hbm,
              slab_v, src_a, dst_a, src_b, dst_b, rows_a, rows_b,
              acc_sh, sem_a, sem_b):
    c = lax.axis_index("c")
    s = lax.axis_index("s")
    wid = c * _NS + s
    pltpu.async_copy(zero_hbm, acc_sh.at[pl.ds(s * _RPT, _RPT)], sem_a)
    pltpu.async_copy(eidx_hbm.at[wid], slab_v, sem_b)
    pltpu.make_async_copy(zero_hbm, acc_sh.at[pl.ds(s * _RPT, _RPT)],
                          sem_a).wait()
    pltpu.make_async_copy(eidx_hbm.at[wid], slab_v, sem_b).wait()
    plsc.subcore_barrier()

    _unpack_chunk(slab_v, 0, src_a, dst_a)
    pltpu.async_copy(h_hbm.at[src_a], rows_a, sem_a)
    _unpack_chunk(slab_v, 1, src_b, dst_b)

    def body(t, carry):
        j = 2 * t
        pltpu.async_copy(h_hbm.at[src_b], rows_b, sem_b)
        pltpu.make_async_copy(h_hbm.at[src_a], rows_a, sem_a).wait()
        pltpu.sync_copy(rows_a, acc_sh.at[dst_a], add=True)
        _unpack_chunk(slab_v, jnp.minimum(j + 2, _NCHUNK - 1), src_a, dst_a)
        pltpu.async_copy(h_hbm.at[src_a], rows_a, sem_a)
        pltpu.make_async_copy(h_hbm.at[src_b], rows_b, sem_b).wait()
        pltpu.sync_copy(rows_b, acc_sh.at[dst_b], add=True)
        _unpack_chunk(slab_v, jnp.minimum(j + 3, _NCHUNK - 1), src_b, dst_b)
        return carry

    lax.fori_loop(0, _NCHUNK // 2, body, 0)
    pltpu.make_async_copy(h_hbm.at[src_a], rows_a, sem_a).wait()
    plsc.subcore_barrier()
    pltpu.sync_copy(acc_sh.at[pl.ds(s * _RPT, _RPT)],
                    out_hbm.at[c, pl.ds(s * _RPT, _RPT)])


@functools.lru_cache(maxsize=None)
def _agg_sc_kernel():
    mesh = plsc.VectorSubcoreMesh(core_axis_name="c", subcore_axis_name="s",
                                  num_cores=_NC, num_subcores=_NS)
    return pl.kernel(
        _agg_body,
        out_type=jax.ShapeDtypeStruct((_NC, _NP, _D), jnp.float32),
        mesh=mesh,
        scratch_types=[
            pltpu.VMEM((_NCHUNK, _CHUNK), jnp.int32),
            pltpu.VMEM((_CHUNK,), jnp.int32),
            pltpu.VMEM((_CHUNK,), jnp.int32),
            pltpu.VMEM((_CHUNK,), jnp.int32),
            pltpu.VMEM((_CHUNK,), jnp.int32),
            pltpu.VMEM((_CHUNK, _D), jnp.float32),
            pltpu.VMEM((_CHUNK, _D), jnp.float32),
            pltpu.VMEM_SHARED((_NP, _D), jnp.float32),
            pltpu.SemaphoreType.DMA,
            pltpu.SemaphoreType.DMA,
        ],
    )


def _full(shape):
    return pl.BlockSpec(shape, lambda i: (0,) * len(shape))


def _rows(shape):
    return pl.BlockSpec(shape, lambda i: (i,) + (0,) * (len(shape) - 1))


def _dot(a, b):
    return jnp.dot(a, b, preferred_element_type=jnp.float32)


def _pre_body(x_ref, wa_ref, ba_ref, wb_ref, o_ref):
    h0 = _dot(x_ref[...], wa_ref[...]) + ba_ref[...]
    o_ref[...] = _dot(h0, wb_ref[...])


def _pre_tc(x, wpre_t, b_pre, w1_t):
    return pl.pallas_call(
        _pre_body,
        grid=(_N // _BLK,),
        in_specs=[_rows((_BLK, _D)), _full((_D, _D)), _full((1, _D)),
                  _full((_D, _D))],
        out_specs=_rows((_BLK, _D)),
        out_shape=jax.ShapeDtypeStruct((_N, _D), jnp.float32),
    )(x, wpre_t, b_pre, w1_t)


def _parts_spec():
    return pl.BlockSpec((_NC, _BLK, _D), lambda i: (0, i, 0))


def _mid_body(p_ref, a_ref, b_ref, w_ref, o_ref):
    h = jnp.maximum(p_ref[...] + a_ref[0] + a_ref[1] + b_ref[...], 0.0)
    o_ref[...] = _dot(h, w_ref[...])


def _mid_tc(p, parts, b1, w2_t):
    return pl.pallas_call(
        _mid_body,
        grid=(_N // _BLK,),
        in_specs=[_rows((_BLK, _D)), _parts_spec(), _full((1, _D)),
                  _full((_D, _D))],
        out_specs=_rows((_BLK, _D)),
        out_shape=jax.ShapeDtypeStruct((_N, _D), jnp.float32),
    )(p, parts, b1, w2_t)


def _out_body(q_ref, a_ref, b_ref, o_ref):
    o_ref[...] = q_ref[...] + a_ref[0] + a_ref[1] + b_ref[...]


def _out_tc(q, parts, b2):
    return pl.pallas_call(
        _out_body,
        grid=(_N // _BLK,),
        in_specs=[_rows((_BLK, _D)), _parts_spec(), _full((1, _D))],
        out_specs=_rows((_BLK, _D)),
        out_shape=jax.ShapeDtypeStruct((_N, _D), jnp.float32),
    )(q, parts, b2)


def kernel(x, edge_index, W_pre, b_pre, W1, b1, W2, b2):
    per_w = _EPW // _NCHUNK
    src = edge_index[0].reshape(_NW, _NCHUNK, per_w)
    dst = edge_index[1].reshape(_NW, _NCHUNK, per_w)
    pad = _CHUNK - per_w
    sent = _N + (jnp.arange(_NW)[:, None, None] * pad
                 + jnp.arange(pad)[None, None, :]) % (_NP - _N)
    src = jnp.pad(src, ((0, 0), (0, 0), (0, pad)))
    dst = jnp.concatenate(
        [dst, jnp.broadcast_to(sent, (_NW, _NCHUNK, pad)).astype(jnp.int32)],
        axis=2)
    eidx = src | (dst << 16)
    zeros = jnp.zeros((_RPT, _D), jnp.float32)

    p = _pre_tc(x, W_pre.T, b_pre.reshape(1, _D), W1.T)
    agg = _agg_sc_kernel()
    parts = agg(p, eidx, zeros)
    q = _mid_tc(p, parts, b1.reshape(1, _D), W2.T)
    parts2 = agg(q, eidx, zeros)
    return _out_tc(q, parts2, b2.reshape(1, _D))


# speedup vs baseline: 3.2524x; 3.2524x over previous
import functools

import jax
import jax.numpy as jnp
from jax import lax
from jax.experimental import pallas as pl
from jax.experimental.pallas import tpu as pltpu
from jax.experimental.pallas import tpu_sc as plsc

_N = 10000
_D = 128
_E = 320000
_NC = 2
_NS = 16
_NW = _NC * _NS
_EPW = _E // _NW
_CHUNK = 80
_NCHUNK = 125
_EPWP = _NCHUNK * _CHUNK
_NP = 10240
_RPT = _NP // _NS

_BLK = 2000

def _unpack_chunk(slab_v, j, src_b, dst_b):
    row = slab_v.at[j]
    for k in range(_CHUNK // 16):
        e = row[pl.ds(16 * k, 16)]
        src_b[pl.ds(16 * k, 16)] = e & 0xFFFF
        dst_b[pl.ds(16 * k, 16)] = lax.shift_right_logical(e, 16)


def _agg_body(h_hbm, eidx_hbm, zero_hbm, out_hbm,
              slab_v, src_a, dst_a, src_b, dst_b, src_c, dst_c,
              rows_a, rows_b, rows_c, acc_sh, sem_a, sem_b, sem_c):
    c = lax.axis_index("c")
    s = lax.axis_index("s")
    wid = c * _NS + s
    pltpu.async_copy(zero_hbm, acc_sh.at[pl.ds(s * _RPT, _RPT)], sem_a)
    pltpu.async_copy(eidx_hbm.at[wid], slab_v, sem_b)
    pltpu.make_async_copy(zero_hbm, acc_sh.at[pl.ds(s * _RPT, _RPT)],
                          sem_a).wait()
    pltpu.make_async_copy(eidx_hbm.at[wid], slab_v, sem_b).wait()
    plsc.subcore_barrier()

    _unpack_chunk(slab_v, 0, src_a, dst_a)
    pltpu.async_copy(h_hbm.at[src_a], rows_a, sem_a)
    _unpack_chunk(slab_v, 1, src_b, dst_b)
    pltpu.async_copy(h_hbm.at[src_b], rows_b, sem_b)
    _unpack_chunk(slab_v, 2, src_c, dst_c)

    def body(t, carry):
        j = 3 * t
        pltpu.async_copy(h_hbm.at[src_c], rows_c, sem_c)
        pltpu.make_async_copy(h_hbm.at[src_a], rows_a, sem_a).wait()
        pltpu.sync_copy(rows_a, acc_sh.at[dst_a], add=True)
        _unpack_chunk(slab_v, jnp.minimum(j + 3, _NCHUNK - 1), src_a, dst_a)
        pltpu.async_copy(h_hbm.at[src_a], rows_a, sem_a)
        pltpu.make_async_copy(h_hbm.at[src_b], rows_b, sem_b).wait()
        pltpu.sync_copy(rows_b, acc_sh.at[dst_b], add=True)
        _unpack_chunk(slab_v, jnp.minimum(j + 4, _NCHUNK - 1), src_b, dst_b)
        pltpu.async_copy(h_hbm.at[src_b], rows_b, sem_b)
        pltpu.make_async_copy(h_hbm.at[src_c], rows_c, sem_c).wait()
        pltpu.sync_copy(rows_c, acc_sh.at[dst_c], add=True)
        _unpack_chunk(slab_v, jnp.minimum(j + 5, _NCHUNK - 1), src_c, dst_c)
        return carry

    lax.fori_loop(0, (_NCHUNK - 2) // 3, body, 0)
    pltpu.make_async_copy(h_hbm.at[src_a], rows_a, sem_a).wait()
    pltpu.sync_copy(rows_a, acc_sh.at[dst_a], add=True)
    pltpu.make_async_copy(h_hbm.at[src_b], rows_b, sem_b).wait()
    pltpu.sync_copy(rows_b, acc_sh.at[dst_b], add=True)
    plsc.subcore_barrier()
    pltpu.sync_copy(acc_sh.at[pl.ds(s * _RPT, _RPT)],
                    out_hbm.at[c, pl.ds(s * _RPT, _RPT)])


@functools.lru_cache(maxsize=None)
def _agg_sc_kernel():
    mesh = plsc.VectorSubcoreMesh(core_axis_name="c", subcore_axis_name="s",
                                  num_cores=_NC, num_subcores=_NS)
    return pl.kernel(
        _agg_body,
        out_type=jax.ShapeDtypeStruct((_NC, _NP, _D), jnp.float32),
        mesh=mesh,
        scratch_types=[
            pltpu.VMEM((_NCHUNK, _CHUNK), jnp.int32),
            pltpu.VMEM((_CHUNK,), jnp.int32),
            pltpu.VMEM((_CHUNK,), jnp.int32),
            pltpu.VMEM((_CHUNK,), jnp.int32),
            pltpu.VMEM((_CHUNK,), jnp.int32),
            pltpu.VMEM((_CHUNK,), jnp.int32),
            pltpu.VMEM((_CHUNK,), jnp.int32),
            pltpu.VMEM((_CHUNK, _D), jnp.float32),
            pltpu.VMEM((_CHUNK, _D), jnp.float32),
            pltpu.VMEM((_CHUNK, _D), jnp.float32),
            pltpu.VMEM_SHARED((_NP, _D), jnp.float32),
            pltpu.SemaphoreType.DMA,
            pltpu.SemaphoreType.DMA,
            pltpu.SemaphoreType.DMA,
        ],
    )


def _full(shape):
    return pl.BlockSpec(shape, lambda i: (0,) * len(shape))


def _rows(shape):
    return pl.BlockSpec(shape, lambda i: (i,) + (0,) * (len(shape) - 1))


def _dot(a, b):
    return jnp.dot(a, b, preferred_element_type=jnp.float32)


def _pre_body(x_ref, wa_ref, ba_ref, wb_ref, o_ref):
    h0 = _dot(x_ref[...], wa_ref[...]) + ba_ref[...]
    o_ref[...] = _dot(h0, wb_ref[...])


def _pre_tc(x, wpre_t, b_pre, w1_t):
    return pl.pallas_call(
        _pre_body,
        grid=(_N // _BLK,),
        in_specs=[_rows((_BLK, _D)), _full((_D, _D)), _full((1, _D)),
                  _full((_D, _D))],
        out_specs=_rows((_BLK, _D)),
        out_shape=jax.ShapeDtypeStruct((_N, _D), jnp.float32),
    )(x, wpre_t, b_pre, w1_t)


def _parts_spec():
    return pl.BlockSpec((_NC, _BLK, _D), lambda i: (0, i, 0))


def _mid_body(p_ref, a_ref, b_ref, w_ref, o_ref):
    h = jnp.maximum(p_ref[...] + a_ref[0] + a_ref[1] + b_ref[...], 0.0)
    o_ref[...] = _dot(h, w_ref[...])


def _mid_tc(p, parts, b1, w2_t):
    return pl.pallas_call(
        _mid_body,
        grid=(_N // _BLK,),
        in_specs=[_rows((_BLK, _D)), _parts_spec(), _full((1, _D)),
                  _full((_D, _D))],
        out_specs=_rows((_BLK, _D)),
        out_shape=jax.ShapeDtypeStruct((_N, _D), jnp.float32),
    )(p, parts, b1, w2_t)


def _out_body(q_ref, a_ref, b_ref, o_ref):
    o_ref[...] = q_ref[...] + a_ref[0] + a_ref[1] + b_ref[...]


def _out_tc(q, parts, b2):
    return pl.pallas_call(
        _out_body,
        grid=(_N // _BLK,),
        in_specs=[_rows((_BLK, _D)), _parts_spec(), _full((1, _D))],
        out_specs=_rows((_BLK, _D)),
        out_shape=jax.ShapeDtypeStruct((_N, _D), jnp.float32),
    )(q, parts, b2)


def kernel(x, edge_index, W_pre, b_pre, W1, b1, W2, b2):
    src = edge_index[0].reshape(_NW, _NCHUNK, _CHUNK)
    dst = edge_index[1].reshape(_NW, _NCHUNK, _CHUNK)
    eidx = src | (dst << 16)
    zeros = jnp.zeros((_RPT, _D), jnp.float32)

    p = _pre_tc(x, W_pre.T, b_pre.reshape(1, _D), W1.T)
    agg = _agg_sc_kernel()
    parts = agg(p, eidx, zeros)
    q = _mid_tc(p, parts, b1.reshape(1, _D), W2.T)
    parts2 = agg(q, eidx, zeros)
    return _out_tc(q, parts2, b2.reshape(1, _D))
